# baseline (speedup 1.0000x reference)
import functools

import jax
import jax.numpy as jnp
from jax import lax
from jax.experimental import pallas as pl
from jax.experimental.pallas import tpu as pltpu
from jax.experimental.pallas import tpu_sc as plsc

NUM_EXPERTS = 64
TOP_K = 2
NUM_TOKENS = 16384
D_MODEL = 1024
N = NUM_TOKENS * TOP_K

NC = 2
NS = 16
W = NC * NS
CH = N // W
NV = CH // 16
TOK = NUM_TOKENS // W
R = 16
NCH = TOK // R
NB = 6
LAG = 3

_mesh = plsc.VectorSubcoreMesh(
    core_axis_name="c", subcore_axis_name="s", num_cores=NC, num_subcores=NS
)


def _wid():
    return lax.axis_index("s") * NC + lax.axis_index("c")


@functools.partial(
    pl.kernel,
    out_type=(
        jax.ShapeDtypeStruct((W, NUM_EXPERTS), jnp.int32),
        jax.ShapeDtypeStruct((N,), jnp.int32),
    ),
    mesh=_mesh,
    compiler_params=pltpu.CompilerParams(needs_layout_passes=False),
    scratch_types=[
        pltpu.VMEM((CH,), jnp.int32),
        pltpu.VMEM((CH,), jnp.int32),
        pltpu.VMEM((CH,), jnp.int32),
        pltpu.VMEM((CH,), jnp.int32),
        pltpu.VMEM((CH,), jnp.int32),
        pltpu.VMEM((NUM_EXPERTS,), jnp.int32),
    ],
)
def _k1(flat_hbm, hist_out, rank_out, keys_v, lrank_v, rank_v, cnt_v,
        loff_v, hist_v):
    wid = _wid()
    base = wid * CH
    iota16 = lax.iota(jnp.int32, 16)
    lane64 = iota16 * 64
    pltpu.sync_copy(flat_hbm.at[pl.ds(base, CH)], keys_v)
    zeros = jnp.zeros((16,), jnp.int32)
    for q in range(CH // 16):
        cnt_v[pl.ds(q * 16, 16)] = zeros
    for s in range(64):
        keys = plsc.load_gather(keys_v, [lane64 + s])
        cidx = lane64 + keys
        cur = plsc.load_gather(cnt_v, [cidx])
        lrank_v[pl.ds(s * 16, 16)] = cur
        plsc.store_scatter(cnt_v, [cidx], cur + 1)
    acc = [jnp.zeros((16,), jnp.int32) for _ in range(4)]
    for l in range(16):
        for q in range(4):
            loff_v[pl.ds(l * 64 + q * 16, 16)] = acc[q]
            acc[q] = acc[q] + cnt_v[pl.ds(l * 64 + q * 16, 16)]
    for q in range(4):
        hist_v[pl.ds(q * 16, 16)] = acc[q]
    for s in range(64):
        keys = plsc.load_gather(keys_v, [lane64 + s])
        lo = plsc.load_gather(loff_v, [lane64 + keys])
        plsc.store_scatter(
            rank_v, [lane64 + s], lrank_v[pl.ds(s * 16, 16)] + lo
        )
    pltpu.sync_copy(hist_v, hist_out.at[wid])
    pltpu.sync_copy(rank_v, rank_out.at[pl.ds(base, CH)])


@functools.partial(
    pl.kernel,
    out_type=(
        jax.ShapeDtypeStruct((N, D_MODEL), jnp.float32),
        jax.ShapeDtypeStruct((NC, N), jnp.int32),
        jax.ShapeDtypeStruct((W * 8, 128), jnp.int32),
        jax.ShapeDtypeStruct((NUM_EXPERTS,), jnp.int32),
    ),
    mesh=_mesh,
    compiler_params=pltpu.CompilerParams(needs_layout_passes=False),
    scratch_types=[
        pltpu.VMEM((CH,), jnp.int32),
        pltpu.VMEM((CH,), jnp.int32),
        pltpu.VMEM((W * NUM_EXPERTS,), jnp.int32),
        pltpu.VMEM((NUM_EXPERTS,), jnp.int32),
        pltpu.VMEM((NUM_EXPERTS,), jnp.int32),
        pltpu.VMEM((CH,), jnp.int32),
        pltpu.VMEM((CH,), jnp.int32),
        pltpu.VMEM((8, 128), jnp.int32),
        pltpu.VMEM((N // NS,), jnp.int32),
        pltpu.VMEM_SHARED((N,), jnp.int32),
        pltpu.VMEM((NB, R, D_MODEL), jnp.float32),
        pltpu.VMEM((NB, 2, R), jnp.int32),
        pltpu.SemaphoreType.DMA((NB,)),
        pltpu.SemaphoreType.DMA((NB,)),
    ],
)
def _k2(hidden_hbm, flat_hbm, hist_hbm, rank_hbm,
        rows_out, sidx_parts_out, iota_out, counts_out,
        keys_v, rank_v, hist_v, start_v, counts_v, pos_v, idsp_v, ids2d,
        zbuf, stage_s, rows_v, idx3, sem_in, sem_out):
    wid = _wid()
    base = wid * CH
    tok0 = wid * TOK
    iota16 = lax.iota(jnp.int32, 16)

    in_descs = []
    for b in range(NB):
        d = pltpu.make_async_copy(
            hidden_hbm.at[pl.ds(tok0 + b * R, R)], rows_v.at[b], sem_in.at[b]
        )
        d.start()
        in_descs.append(d)

    pltpu.sync_copy(flat_hbm.at[pl.ds(base, CH)], keys_v)
    pltpu.sync_copy(rank_hbm.at[pl.ds(base, CH)], rank_v)
    pltpu.sync_copy(hist_hbm, hist_v)

    widv = lax.broadcast_in_dim(wid, (16,), ())
    carry = jnp.int32(0)
    for q in range(NUM_EXPERTS // 16):
        acc = jnp.zeros((16,), jnp.int32)
        part = jnp.zeros((16,), jnp.int32)
        for w2 in range(W):
            part = jnp.where(widv == w2, acc, part)
            acc = acc + hist_v[pl.ds(w2 * NUM_EXPERTS + q * 16, 16)]
        counts_v[pl.ds(q * 16, 16)] = acc
        excl = plsc.cumsum(acc) - acc
        start_v[pl.ds(q * 16, 16)] = (
            excl + part + lax.broadcast_in_dim(carry, (16,), ())
        )
        carry = carry + jnp.sum(acc)

    for v in range(64):
        keys = keys_v[pl.ds(v * 16, 16)]
        st = plsc.load_gather(start_v, [keys])
        ids = base + v * 16 + iota16
        pos_v[pl.ds(v * 16, 16)] = st + rank_v[pl.ds(v * 16, 16)]
        idsp_v[pl.ds(v * 16, 16)] = ids + 1
        ids2d[v // 8, pl.ds((v % 8) * 16, 16)] = ids

    pltpu.sync_copy(ids2d, iota_out.at[pl.ds(wid * 8, 8)])

    @pl.when(wid == 0)
    def _():
        pltpu.sync_copy(counts_v, counts_out)

    sid = lax.axis_index("s")
    cid = lax.axis_index("c")
    SL = N // NS
    zeros16 = jnp.zeros((16,), jnp.int32)
    for q in range(SL // 16):
        zbuf[pl.ds(q * 16, 16)] = zeros16
    pltpu.sync_copy(zbuf, stage_s.at[pl.ds(sid * SL, SL)])
    plsc.subcore_barrier()
    pltpu.sync_copy(idsp_v, stage_s.at[pos_v])
    plsc.subcore_barrier()
    pltpu.sync_copy(
        stage_s.at[pl.ds(sid * SL, SL)],
        sidx_parts_out.at[cid, pl.ds(sid * SL, SL)],
    )

    out_descs = []
    for ch in range(NCH):
        b = ch % NB
        in_descs[ch].wait()
        e0 = ch * 2 * R + 2 * iota16
        idx3[b, 0] = plsc.load_gather(pos_v, [e0])
        idx3[b, 1] = plsc.load_gather(pos_v, [e0 + 1])
        dA = pltpu.make_async_copy(
            rows_v.at[b], rows_out.at[idx3.at[b, 0]], sem_out.at[b]
        )
        dB = pltpu.make_async_copy(
            rows_v.at[b], rows_out.at[idx3.at[b, 1]], sem_out.at[b]
        )
        dA.start()
        dB.start()
        out_descs.append((dA, dB))
        if ch >= LAG:
            for p in out_descs[ch - LAG]:
                p.wait()
            if ch + (NB - LAG) < NCH:
                nxt = pltpu.make_async_copy(
                    hidden_hbm.at[pl.ds(tok0 + (ch + NB - LAG) * R, R)],
                    rows_v.at[(ch + NB - LAG) % NB],
                    sem_in.at[(ch + NB - LAG) % NB],
                )
                nxt.start()
                in_descs.append(nxt)

    for ch in range(NCH - LAG, NCH):
        for p in out_descs[ch]:
            p.wait()


def kernel(hidden_states, routing_indices):
    flat = routing_indices.reshape(-1).astype(jnp.int32)
    hist, rank = _k1(flat)
    rows, sidx_parts, iota2d, counts = _k2(
        hidden_states, flat, hist.reshape(-1), rank
    )
    sortidx = sidx_parts[0] + sidx_parts[1] - 1
    return rows, sortidx, iota2d.reshape(-1), counts


# speedup vs baseline: 1.0608x; 1.0608x over previous
import functools

import jax
import jax.numpy as jnp
from jax import lax
from jax.experimental import pallas as pl
from jax.experimental.pallas import tpu as pltpu
from jax.experimental.pallas import tpu_sc as plsc

NUM_EXPERTS = 64
TOP_K = 2
NUM_TOKENS = 16384
D_MODEL = 1024
N = NUM_TOKENS * TOP_K

NC = 2
NS = 16
W = NC * NS
CH = N // W
TOK = NUM_TOKENS // W
R = 16
NCH = TOK // R
NB = 6
LAG = 3
SL = N // NS

_mesh = plsc.VectorSubcoreMesh(
    core_axis_name="c", subcore_axis_name="s", num_cores=NC, num_subcores=NS
)


@functools.partial(
    pl.kernel,
    out_type=(
        jax.ShapeDtypeStruct((N, D_MODEL), jnp.float32),
        jax.ShapeDtypeStruct((NC, N), jnp.int32),
        jax.ShapeDtypeStruct((W * 8, 128), jnp.int32),
        jax.ShapeDtypeStruct((NUM_EXPERTS,), jnp.int32),
    ),
    mesh=_mesh,
    compiler_params=pltpu.CompilerParams(needs_layout_passes=False),
    scratch_types=[
        pltpu.VMEM((CH,), jnp.int32),
        pltpu.VMEM((CH,), jnp.int32),
        pltpu.VMEM((CH,), jnp.int32),
        pltpu.VMEM((CH,), jnp.int32),
        pltpu.VMEM((CH,), jnp.int32),
        pltpu.VMEM((CH,), jnp.int32),
        pltpu.VMEM((NUM_EXPERTS,), jnp.int32),
        pltpu.VMEM((NUM_EXPERTS,), jnp.int32),
        pltpu.VMEM((W * NUM_EXPERTS,), jnp.int32),
        pltpu.VMEM((NUM_EXPERTS,), jnp.int32),
        pltpu.VMEM((NUM_EXPERTS,), jnp.int32),
        pltpu.VMEM((CH,), jnp.int32),
        pltpu.VMEM((CH,), jnp.int32),
        pltpu.VMEM((8, 128), jnp.int32),
        pltpu.VMEM((SL,), jnp.int32),
        pltpu.VMEM_SHARED((W * NUM_EXPERTS,), jnp.int32),
        pltpu.VMEM_SHARED((N,), jnp.int32),
        pltpu.VMEM((NB, R, D_MODEL), jnp.float32),
        pltpu.VMEM((NB, 2, R), jnp.int32),
        pltpu.SemaphoreType.DMA((NB,)),
        pltpu.SemaphoreType.DMA((NB,)),
    ],
)
def _k(hidden_hbm, flat_hbm,
       rows_out, sidx_parts_out, iota_out, counts_out,
       keys_v, keys2_v, lrank_v, rank_v, cnt_v, loff_v, hist_v, hist2_v,
       allh_v, start_v, counts_v, pos_v, idsp_v, ids2d, zbuf,
       histshare_s, stage_s, rows_v, idx3, sem_in, sem_out):
    cid = lax.axis_index("c")
    sid = lax.axis_index("s")
    wid = sid * NC + cid
    mw = sid * NC + (NC - 1) - cid
    base = wid * CH
    tok0 = wid * TOK
    iota16 = lax.iota(jnp.int32, 16)
    lane64 = iota16 * 64
    zeros16 = jnp.zeros((16,), jnp.int32)

    in_descs = []
    for b in range(NB):
        d = pltpu.make_async_copy(
            hidden_hbm.at[pl.ds(tok0 + b * R, R)], rows_v.at[b], sem_in.at[b]
        )
        d.start()
        in_descs.append(d)

    pltpu.sync_copy(flat_hbm.at[pl.ds(base, CH)], keys_v)
    pltpu.sync_copy(flat_hbm.at[pl.ds(mw * CH, CH)], keys2_v)
    for q in range(CH // 16):
        cnt_v[pl.ds(q * 16, 16)] = zeros16
    for s in range(64):
        keys = plsc.load_gather(keys_v, [lane64 + s])
        cidx = lane64 + keys
        cur = plsc.load_gather(cnt_v, [cidx])
        lrank_v[pl.ds(s * 16, 16)] = cur
        plsc.store_scatter(cnt_v, [cidx], cur + 1)
    acc = [jnp.zeros((16,), jnp.int32) for _ in range(4)]
    for l in range(16):
        for q in range(4):
            loff_v[pl.ds(l * 64 + q * 16, 16)] = acc[q]
            acc[q] = acc[q] + cnt_v[pl.ds(l * 64 + q * 16, 16)]
    for q in range(4):
        hist_v[pl.ds(q * 16, 16)] = acc[q]
    for s in range(64):
        keys = plsc.load_gather(keys_v, [lane64 + s])
        lo = plsc.load_gather(loff_v, [lane64 + keys])
        plsc.store_scatter(
            rank_v, [lane64 + s], lrank_v[pl.ds(s * 16, 16)] + lo
        )
    for q in range(CH // 16):
        cnt_v[pl.ds(q * 16, 16)] = zeros16
    for s in range(64):
        keys = plsc.load_gather(keys2_v, [lane64 + s])
        cidx = lane64 + keys
        cur = plsc.load_gather(cnt_v, [cidx])
        plsc.store_scatter(cnt_v, [cidx], cur + 1)
    acc = [jnp.zeros((16,), jnp.int32) for _ in range(4)]
    for l in range(16):
        for q in range(4):
            acc[q] = acc[q] + cnt_v[pl.ds(l * 64 + q * 16, 16)]
    for q in range(4):
        hist2_v[pl.ds(q * 16, 16)] = acc[q]

    pltpu.sync_copy(hist_v, histshare_s.at[pl.ds(wid * NUM_EXPERTS,
                                                 NUM_EXPERTS)])
    pltpu.sync_copy(hist2_v, histshare_s.at[pl.ds(mw * NUM_EXPERTS,
                                                  NUM_EXPERTS)])
    plsc.subcore_barrier()
    pltpu.sync_copy(histshare_s, allh_v)

    widv = lax.broadcast_in_dim(wid, (16,), ())
    carry = jnp.int32(0)
    for q in range(NUM_EXPERTS // 16):
        acc = jnp.zeros((16,), jnp.int32)
        part = jnp.zeros((16,), jnp.int32)
        for w2 in range(W):
            part = jnp.where(widv == w2, acc, part)
            acc = acc + allh_v[pl.ds(w2 * NUM_EXPERTS + q * 16, 16)]
        counts_v[pl.ds(q * 16, 16)] = acc
        excl = plsc.cumsum(acc) - acc
        start_v[pl.ds(q * 16, 16)] = (
            excl + part + lax.broadcast_in_dim(carry, (16,), ())
        )
        carry = carry + jnp.sum(acc)

    for v in range(64):
        keys = keys_v[pl.ds(v * 16, 16)]
        st = plsc.load_gather(start_v, [keys])
        ids = base + v * 16 + iota16
        pos_v[pl.ds(v * 16, 16)] = st + rank_v[pl.ds(v * 16, 16)]
        idsp_v[pl.ds(v * 16, 16)] = ids + 1
        ids2d[v // 8, pl.ds((v % 8) * 16, 16)] = ids

    pltpu.sync_copy(ids2d, iota_out.at[pl.ds(wid * 8, 8)])

    @pl.when(wid == 0)
    def _():
        pltpu.sync_copy(counts_v, counts_out)

    for q in range(SL // 16):
        zbuf[pl.ds(q * 16, 16)] = zeros16
    pltpu.sync_copy(zbuf, stage_s.at[pl.ds(sid * SL, SL)])
    plsc.subcore_barrier()
    pltpu.sync_copy(idsp_v, stage_s.at[pos_v])
    plsc.subcore_barrier()
    pltpu.sync_copy(
        stage_s.at[pl.ds(sid * SL, SL)],
        sidx_parts_out.at[cid, pl.ds(sid * SL, SL)],
    )

    out_descs = []
    for ch in range(NCH):
        b = ch % NB
        in_descs[ch].wait()
        e0 = ch * 2 * R + 2 * iota16
        idx3[b, 0] = plsc.load_gather(pos_v, [e0])
        idx3[b, 1] = plsc.load_gather(pos_v, [e0 + 1])
        dA = pltpu.make_async_copy(
            rows_v.at[b], rows_out.at[idx3.at[b, 0]], sem_out.at[b]
        )
        dB = pltpu.make_async_copy(
            rows_v.at[b], rows_out.at[idx3.at[b, 1]], sem_out.at[b]
        )
        dA.start()
        dB.start()
        out_descs.append((dA, dB))
        if ch >= LAG:
            for p in out_descs[ch - LAG]:
                p.wait()
            if ch + (NB - LAG) < NCH:
                nxt = pltpu.make_async_copy(
                    hidden_hbm.at[pl.ds(tok0 + (ch + NB - LAG) * R, R)],
                    rows_v.at[(ch + NB - LAG) % NB],
                    sem_in.at[(ch + NB - LAG) % NB],
                )
                nxt.start()
                in_descs.append(nxt)

    for ch in range(NCH - LAG, NCH):
        for p in out_descs[ch]:
            p.wait()


def kernel(hidden_states, routing_indices):
    flat = routing_indices.reshape(-1).astype(jnp.int32)
    rows, sidx_parts, iota2d, counts = _k(hidden_states, flat)
    sortidx = sidx_parts[0] + sidx_parts[1] - 1
    return rows, sortidx, iota2d.reshape(-1), counts


# speedup vs baseline: 1.0729x; 1.0114x over previous
import functools

import jax
import jax.numpy as jnp
from jax import lax
from jax.experimental import pallas as pl
from jax.experimental.pallas import tpu as pltpu
from jax.experimental.pallas import tpu_sc as plsc

NUM_EXPERTS = 64
TOP_K = 2
NUM_TOKENS = 16384
D_MODEL = 1024
N = NUM_TOKENS * TOP_K

NC = 2
NS = 16
W = NC * NS
CH = N // W
TOK = NUM_TOKENS // W
R = 16
NCH = TOK // R
NB = 6
LAG = 3
SL = N // NS

_mesh = plsc.VectorSubcoreMesh(
    core_axis_name="c", subcore_axis_name="s", num_cores=NC, num_subcores=NS
)


@functools.partial(
    pl.kernel,
    out_type=(
        jax.ShapeDtypeStruct((N, D_MODEL), jnp.float32),
        jax.ShapeDtypeStruct((NC, N), jnp.int32),
        jax.ShapeDtypeStruct((W * 8, 128), jnp.int32),
        jax.ShapeDtypeStruct((NUM_EXPERTS,), jnp.int32),
    ),
    mesh=_mesh,
    compiler_params=pltpu.CompilerParams(needs_layout_passes=False),
    scratch_types=[
        pltpu.VMEM((CH,), jnp.int32),
        pltpu.VMEM((CH,), jnp.int32),
        pltpu.VMEM((CH,), jnp.int32),
        pltpu.VMEM((CH,), jnp.int32),
        pltpu.VMEM((CH,), jnp.int32),
        pltpu.VMEM((CH,), jnp.int32),
        pltpu.VMEM((NUM_EXPERTS,), jnp.int32),
        pltpu.VMEM((NUM_EXPERTS,), jnp.int32),
        pltpu.VMEM((W * NUM_EXPERTS,), jnp.int32),
        pltpu.VMEM((NUM_EXPERTS,), jnp.int32),
        pltpu.VMEM((NUM_EXPERTS,), jnp.int32),
        pltpu.VMEM((CH,), jnp.int32),
        pltpu.VMEM((CH,), jnp.int32),
        pltpu.VMEM((8, 128), jnp.int32),
        pltpu.VMEM((SL,), jnp.int32),
        pltpu.VMEM_SHARED((W * NUM_EXPERTS,), jnp.int32),
        pltpu.VMEM_SHARED((N,), jnp.int32),
        pltpu.VMEM((NB, R, D_MODEL), jnp.float32),
        pltpu.VMEM((NB, 2, R), jnp.int32),
        pltpu.SemaphoreType.DMA((NB,)),
        pltpu.SemaphoreType.DMA((NB,)),
    ],
)
def _k(hidden_hbm, flat_hbm,
       rows_out, sidx_parts_out, iota_out, counts_out,
       keys_v, keys2_v, lrank_v, rank_v, cnt_v, loff_v, hist_v, hist2_v,
       allh_v, start_v, counts_v, pos_v, idsp_v, ids2d, zbuf,
       histshare_s, stage_s, rows_v, idx3, sem_in, sem_out):
    cid = lax.axis_index("c")
    sid = lax.axis_index("s")
    wid = sid * NC + cid
    mw = sid * NC + (NC - 1) - cid
    base = wid * CH
    tok0 = wid * TOK
    iota16 = lax.iota(jnp.int32, 16)
    lane64 = iota16 * 64
    zeros16 = jnp.zeros((16,), jnp.int32)

    in_descs = []
    for b in range(NB):
        d = pltpu.make_async_copy(
            hidden_hbm.at[pl.ds(tok0 + b * R, R)], rows_v.at[b], sem_in.at[b]
        )
        d.start()
        in_descs.append(d)

    pltpu.sync_copy(flat_hbm.at[pl.ds(base, CH)], keys_v)
    pltpu.sync_copy(flat_hbm.at[pl.ds(mw * CH, CH)], keys2_v)
    for q in range(CH // 16):
        cnt_v[pl.ds(q * 16, 16)] = zeros16
    for s in range(64):
        keys = plsc.load_gather(keys_v, [lane64 + s])
        cidx = lane64 + keys
        cur = plsc.load_gather(cnt_v, [cidx])
        lrank_v[pl.ds(s * 16, 16)] = cur
        plsc.store_scatter(cnt_v, [cidx], cur + 1)
    acc = [jnp.zeros((16,), jnp.int32) for _ in range(4)]
    for l in range(16):
        for q in range(4):
            loff_v[pl.ds(l * 64 + q * 16, 16)] = acc[q]
            acc[q] = acc[q] + cnt_v[pl.ds(l * 64 + q * 16, 16)]
    for q in range(4):
        hist_v[pl.ds(q * 16, 16)] = acc[q]
    for s in range(64):
        keys = plsc.load_gather(keys_v, [lane64 + s])
        lo = plsc.load_gather(loff_v, [lane64 + keys])
        plsc.store_scatter(
            rank_v, [lane64 + s], lrank_v[pl.ds(s * 16, 16)] + lo
        )
    for q in range(CH // 16):
        cnt_v[pl.ds(q * 16, 16)] = zeros16
    for s in range(64):
        keys = plsc.load_gather(keys2_v, [lane64 + s])
        cidx = lane64 + keys
        cur = plsc.load_gather(cnt_v, [cidx])
        plsc.store_scatter(cnt_v, [cidx], cur + 1)
    acc = [jnp.zeros((16,), jnp.int32) for _ in range(4)]
    for l in range(16):
        for q in range(4):
            acc[q] = acc[q] + cnt_v[pl.ds(l * 64 + q * 16, 16)]
    for q in range(4):
        hist2_v[pl.ds(q * 16, 16)] = acc[q]

    pltpu.sync_copy(hist_v, histshare_s.at[pl.ds(wid * NUM_EXPERTS,
                                                 NUM_EXPERTS)])
    pltpu.sync_copy(hist2_v, histshare_s.at[pl.ds(mw * NUM_EXPERTS,
                                                  NUM_EXPERTS)])
    plsc.subcore_barrier()
    pltpu.sync_copy(histshare_s, allh_v)

    widv = lax.broadcast_in_dim(wid, (16,), ())
    carry = jnp.int32(0)
    for q in range(NUM_EXPERTS // 16):
        acc = jnp.zeros((16,), jnp.int32)
        part = jnp.zeros((16,), jnp.int32)
        for w2 in range(W):
            part = jnp.where(widv == w2, acc, part)
            acc = acc + allh_v[pl.ds(w2 * NUM_EXPERTS + q * 16, 16)]
        counts_v[pl.ds(q * 16, 16)] = acc
        excl = plsc.cumsum(acc) - acc
        start_v[pl.ds(q * 16, 16)] = (
            excl + part + lax.broadcast_in_dim(carry, (16,), ())
        )
        carry = carry + jnp.sum(acc)

    for v in range(64):
        keys = keys_v[pl.ds(v * 16, 16)]
        st = plsc.load_gather(start_v, [keys])
        ids = base + v * 16 + iota16
        pos_v[pl.ds(v * 16, 16)] = st + rank_v[pl.ds(v * 16, 16)]
        idsp_v[pl.ds(v * 16, 16)] = ids + 1
        ids2d[v // 8, pl.ds((v % 8) * 16, 16)] = ids

    out_descs = []

    def _chunk(ch):
        b = ch % NB
        in_descs[ch].wait()
        e0 = ch * 2 * R + 2 * iota16
        idx3[b, 0] = plsc.load_gather(pos_v, [e0])
        idx3[b, 1] = plsc.load_gather(pos_v, [e0 + 1])
        dA = pltpu.make_async_copy(
            rows_v.at[b], rows_out.at[idx3.at[b, 0]], sem_out.at[b]
        )
        dB = pltpu.make_async_copy(
            rows_v.at[b], rows_out.at[idx3.at[b, 1]], sem_out.at[b]
        )
        dA.start()
        dB.start()
        out_descs.append((dA, dB))
        if ch >= LAG:
            for p in out_descs[ch - LAG]:
                p.wait()
            if ch + (NB - LAG) < NCH:
                nxt = pltpu.make_async_copy(
                    hidden_hbm.at[pl.ds(tok0 + (ch + NB - LAG) * R, R)],
                    rows_v.at[(ch + NB - LAG) % NB],
                    sem_in.at[(ch + NB - LAG) % NB],
                )
                nxt.start()
                in_descs.append(nxt)

    for ch in range(LAG):
        _chunk(ch)

    for q in range(SL // 16):
        zbuf[pl.ds(q * 16, 16)] = zeros16
    pltpu.sync_copy(zbuf, stage_s.at[pl.ds(sid * SL, SL)])
    plsc.subcore_barrier()
    pltpu.sync_copy(idsp_v, stage_s.at[pos_v])
    plsc.subcore_barrier()
    pltpu.sync_copy(
        stage_s.at[pl.ds(sid * SL, SL)],
        sidx_parts_out.at[cid, pl.ds(sid * SL, SL)],
    )

    pltpu.sync_copy(ids2d, iota_out.at[pl.ds(wid * 8, 8)])

    @pl.when(wid == 0)
    def _():
        pltpu.sync_copy(counts_v, counts_out)

    for ch in range(LAG, NCH):
        _chunk(ch)

    for ch in range(NCH - LAG, NCH):
        for p in out_descs[ch]:
            p.wait()


def kernel(hidden_states, routing_indices):
    flat = routing_indices.reshape(-1).astype(jnp.int32)
    rows, sidx_parts, iota2d, counts = _k(hidden_states, flat)
    sortidx = sidx_parts[0] + sidx_parts[1] - 1
    return rows, sortidx, iota2d.reshape(-1), counts
